# Initial kernel scaffold; baseline (speedup 1.0000x reference)
import jax
import jax.numpy as jnp
from jax.experimental import pallas as pl


def kernel(img, W_loc, b_loc, W_cls, b_cls, W_val, b_val, dboxes_xywh):
    raise NotImplementedError("write your pallas kernel here")


# speedup vs baseline: 11.6933x; 11.6933x over previous
import functools

import jax
import jax.numpy as jnp
from jax.experimental import pallas as pl
from jax.experimental.pallas import tpu as pltpu

N = 20000
C = 3
MAX_NUM = 200
NMS_TH = 0.5
CONF_TH = 0.05
D = 192
SCALE_XY = 0.1
SCALE_WH = 0.2

_ROWS = 8
_COLS = N // _ROWS


def _matvec_kernel(feat_ref, wl_ref, ploc_ref):
    ploc_ref[:] = jax.lax.dot(feat_ref[:], wl_ref[:])


def _nms_kernel(feat_ref, wval_ref, bval_ref, ploc_ref, pcls_ref, dbox_ref,
                pvals_ref, abw_b_ref, pbw_b_ref, abw_s_ref, pbw_s_ref,
                sA_ref, sB_ref, L_ref, T_ref, R_ref, B_ref, A2_ref):
    v = jax.lax.dot(feat_ref[:], wval_ref[:]) + bval_ref[:]
    vm = jnp.max(v, axis=-1, keepdims=True)
    ve = jnp.exp(v - vm)
    pvals_ref[:] = ve / jnp.sum(ve, axis=-1, keepdims=True)

    dx = dbox_ref[0]
    dy = dbox_ref[1]
    dw = dbox_ref[2]
    dh = dbox_ref[3]
    x = (ploc_ref[0] * SCALE_XY) * dw + dx
    y = (ploc_ref[1] * SCALE_XY) * dh + dy
    w = jnp.exp(jnp.clip(ploc_ref[2] * SCALE_WH, -10.0, 10.0)) * dw
    h = jnp.exp(jnp.clip(ploc_ref[3] * SCALE_WH, -10.0, 10.0)) * dh
    L_ref[:] = x - 0.5 * w
    T_ref[:] = y - 0.5 * h
    R_ref[:] = x + 0.5 * w
    B_ref[:] = y + 0.5 * h
    A2_ref[:] = jnp.maximum(R_ref[:] - L_ref[:], 0.0) * jnp.maximum(
        B_ref[:] - T_ref[:], 0.0)

    p1 = pcls_ref[0]
    p2 = pcls_ref[1]
    sA_ref[:] = jnp.where(p1 > CONF_TH, p1, -1.0)
    sB_ref[:] = jnp.where(p2 > CONF_TH, p2, -1.0)

    lin = (jax.lax.broadcasted_iota(jnp.int32, (_ROWS, _COLS), 0) * _COLS
           + jax.lax.broadcasted_iota(jnp.int32, (_ROWS, _COLS), 1))
    big = jnp.int32(2**30)

    def one_class(k, s_ref, b_out_ref, s_out_ref):
        s = s_ref[:]
        m = jnp.max(s)
        idx = jnp.min(jnp.where(s == m, lin, big))
        wmask = lin == idx
        valid = m > 0.0
        sl = jnp.sum(jnp.where(wmask, L_ref[:], 0.0))
        st = jnp.sum(jnp.where(wmask, T_ref[:], 0.0))
        sr = jnp.sum(jnp.where(wmask, R_ref[:], 0.0))
        sb = jnp.sum(jnp.where(wmask, B_ref[:], 0.0))
        ltx = jnp.maximum(sl, L_ref[:])
        lty = jnp.maximum(st, T_ref[:])
        rbx = jnp.minimum(sr, R_ref[:])
        rby = jnp.minimum(sb, B_ref[:])
        inter = jnp.maximum(rbx - ltx, 0.0) * jnp.maximum(rby - lty, 0.0)
        a1 = jnp.maximum(sr - sl, 0.0) * jnp.maximum(sb - st, 0.0)
        iou = inter / (a1 + A2_ref[:] - inter + 1e-9)
        s_new = jnp.where(iou > NMS_TH, -1.0, s)
        s_new = jnp.where(wmask, -1.0, s_new)
        s_ref[:] = jnp.where(valid, s_new, s)
        zero = jnp.float32(0.0)
        row = jnp.concatenate(
            [jnp.where(valid, sl, zero).reshape(1, 1),
             jnp.where(valid, st, zero).reshape(1, 1),
             jnp.where(valid, sr, zero).reshape(1, 1),
             jnp.where(valid, sb, zero).reshape(1, 1)], axis=1)
        b_out_ref[pl.ds(k, 1), :] = row
        s_out_ref[pl.ds(k, 1), :] = jnp.where(valid, m, zero).reshape(1, 1)

    def body(k, carry):
        one_class(k, sA_ref, abw_b_ref, abw_s_ref)
        one_class(k, sB_ref, pbw_b_ref, pbw_s_ref)
        return carry

    jax.lax.fori_loop(0, MAX_NUM, body, jnp.int32(0))


def kernel(img, W_loc, b_loc, W_cls, b_cls, W_val, b_val, dboxes_xywh):
    x = img.astype(jnp.float32).reshape(1, 3, 8, 64, 8, 64).mean(axis=(3, 5))
    feat = x.reshape(1, D)

    grid = 16
    bl = 5120
    ploc = pl.pallas_call(
        _matvec_kernel,
        grid=(grid,),
        in_specs=[
            pl.BlockSpec((1, D), lambda i: (0, 0)),
            pl.BlockSpec((D, bl), lambda i: (0, i)),
        ],
        out_specs=pl.BlockSpec((1, bl), lambda i: (0, i)),
        out_shape=jax.ShapeDtypeStruct((1, 4 * N), jnp.float32),
    )(feat, W_loc)
    ploc = (ploc + b_loc.reshape(1, 4 * N)).reshape(4, _ROWS, _COLS)

    plabels = (feat @ W_cls + b_cls).reshape(1, C, N)
    probs = jax.nn.softmax(jnp.transpose(plabels, (0, 2, 1))[0], axis=-1)
    pcls = probs.T[1:].reshape(2, _ROWS, _COLS)
    dbox = dboxes_xywh.T.reshape(4, _ROWS, _COLS)

    outs = pl.pallas_call(
        _nms_kernel,
        out_shape=[
            jax.ShapeDtypeStruct((1, 2), jnp.float32),
            jax.ShapeDtypeStruct((MAX_NUM, 4), jnp.float32),
            jax.ShapeDtypeStruct((MAX_NUM, 4), jnp.float32),
            jax.ShapeDtypeStruct((MAX_NUM, 1), jnp.float32),
            jax.ShapeDtypeStruct((MAX_NUM, 1), jnp.float32),
        ],
        scratch_shapes=[pltpu.VMEM((_ROWS, _COLS), jnp.float32)
                        for _ in range(7)],
    )(feat, W_val, b_val.reshape(1, 2), ploc, pcls, dbox)
    pvals, abw_b, pbw_b, abw_s, pbw_s = outs
    return (pvals, abw_b, pbw_b, abw_s.reshape(MAX_NUM), pbw_s.reshape(MAX_NUM))
